# baseline (speedup 1.0000x reference)
import functools

import jax
import jax.numpy as jnp
from jax.experimental import pallas as pl
from jax.experimental.pallas import tpu as pltpu

RADIUS = 2
TOP_K = 4096
KS = 2 * RADIUS + 1
TEMP = 0.1
H = 512
W = 512
B = 4


def _mp5(x):
    h, w = x.shape
    minf_c = jnp.full((h, RADIUS), -jnp.inf, x.dtype)
    xp = jnp.concatenate([minf_c, x, minf_c], axis=1)
    m = xp[:, 0:w]
    for i in range(1, KS):
        m = jnp.maximum(m, xp[:, i:i + w])
    minf_r = jnp.full((RADIUS, w), -jnp.inf, x.dtype)
    vp = jnp.concatenate([minf_r, m, minf_r], axis=0)
    m = vp[0:h, :]
    for i in range(1, KS):
        m = jnp.maximum(m, vp[i:i + h, :])
    return m


def _nms_body(x_ref, nms_ref):
    x = x_ref[0, 0]
    max_mask = x == _mp5(x)
    zeros = jnp.zeros_like(x)
    for _ in range(2):
        supp_mask = _mp5(max_mask.astype(x.dtype)) > 0
        supp_scores = jnp.where(supp_mask, zeros, x)
        new_max = (supp_scores == _mp5(supp_scores)) & (~supp_mask)
        max_mask = max_mask | new_max
    nms = jnp.where(max_mask, x, zeros)
    ri = jax.lax.broadcasted_iota(jnp.int32, x.shape, 0)
    ci = jax.lax.broadcasted_iota(jnp.int32, x.shape, 1)
    inb = (ri >= RADIUS) & (ri < H - RADIUS) & (ci >= RADIUS) & (ci < W - RADIUS)
    nms_ref[0] = jnp.where(inb, nms, zeros)


@jax.jit
def _nms_pallas(scores_map):
    return pl.pallas_call(
        _nms_body,
        grid=(B,),
        in_specs=[pl.BlockSpec((1, 1, H, W), lambda b: (b, 0, 0, 0))],
        out_specs=pl.BlockSpec((1, H, W), lambda b: (b, 0, 0)),
        out_shape=jax.ShapeDtypeStruct((B, H, W), scores_map.dtype),
    )(scores_map)


def _hw_grid_host(r, dt):
    import numpy as np
    x = np.linspace(-r, r, 2 * r + 1)
    ii, jj = np.meshgrid(x, x, indexing='ij')
    return jnp.asarray(
        np.stack([jj.reshape(-1), ii.reshape(-1)], axis=1), dtype=dt)


def _grid_sample(img, xy):
    h, w = img.shape
    px = (xy[:, 0] + 1.0) * 0.5 * (w - 1)
    py = (xy[:, 1] + 1.0) * 0.5 * (h - 1)
    x0 = jnp.floor(px); y0 = jnp.floor(py)
    x1 = x0 + 1.0; y1 = y0 + 1.0
    wx1 = px - x0; wx0 = 1.0 - wx1
    wy1 = py - y0; wy0 = 1.0 - wy1
    def gat(xi, yi):
        valid = (xi >= 0) & (xi <= w - 1) & (yi >= 0) & (yi <= h - 1)
        xc = jnp.clip(xi, 0, w - 1).astype(jnp.int32)
        yc = jnp.clip(yi, 0, h - 1).astype(jnp.int32)
        return jnp.where(valid, img[yc, xc], 0.0)
    return (gat(x0, y0) * wx0 * wy0 + gat(x1, y0) * wx1 * wy0
            + gat(x0, y1) * wx0 * wy1 + gat(x1, y1) * wx1 * wy1)


@jax.jit
def kernel(scores_map):
    dt = scores_map.dtype
    nms = _nms_pallas(scores_map)
    flat = nms.reshape(B, -1)
    N = H * W
    CAP = 32768
    mask = flat > 0.0
    cnt = mask.sum(axis=1)
    ok = jnp.all((cnt >= TOP_K) & (cnt <= CAP))

    def _fast(f):
        m = f > 0.0
        pos = jnp.cumsum(m.astype(jnp.int32), axis=1)
        dest = jnp.where(m, pos - 1, CAP)
        iota = jnp.broadcast_to(jnp.arange(N, dtype=jnp.int32)[None], (B, N))
        comp_idx = jnp.zeros((B, CAP), jnp.int32).at[
            jnp.arange(B)[:, None], dest].set(iota, mode='drop')
        comp_val = jnp.take_along_axis(f, comp_idx, axis=1)
        _, tki = jax.lax.top_k(comp_val, TOP_K)
        return jnp.take_along_axis(comp_idx, tki, axis=1)

    indices = jax.lax.cond(
        ok, _fast, lambda f: jax.lax.top_k(f, TOP_K)[1], flat)

    r = RADIUS
    padded = jnp.pad(scores_map[:, 0], ((0, 0), (r, r), (r, r)))
    import numpy as np
    oy, ox = np.meshgrid(np.arange(KS), np.arange(KS), indexing='ij')
    oy = jnp.asarray(oy.reshape(-1)); ox = jnp.asarray(ox.reshape(-1))
    hw_grid = _hw_grid_host(r, dt)
    wh = jnp.array([W - 1, H - 1], dtype=dt)

    def per_image(padded_b, img_b, idx):
        rows = idx // W
        cols = idx % W
        patch_scores = padded_b[rows[:, None] + oy[None, :],
                                cols[:, None] + ox[None, :]]
        kp_nms = jnp.stack([cols.astype(dt), rows.astype(dt)], axis=1)
        max_v = patch_scores.max(axis=1)[:, None]
        x_exp = jnp.exp((patch_scores - max_v) / TEMP)
        ssum = x_exp.sum(axis=1)[:, None]
        xy_residual = (x_exp @ hw_grid) / ssum
        dist2 = jnp.sum(((hw_grid[None, :, :] - xy_residual[:, None, :]) / r) ** 2,
                        axis=-1)
        disp = (x_exp * dist2).sum(axis=1) / ssum[:, 0]
        kp = (kp_nms + xy_residual) / wh * 2.0 - 1.0
        sc = _grid_sample(img_b, kp)
        return kp, disp, sc

    return jax.vmap(per_image)(padded, scores_map[:, 0], indices)


# speedup vs baseline: 2.0561x; 2.0561x over previous
import functools

import jax
import jax.numpy as jnp
from jax.experimental import pallas as pl
from jax.experimental.pallas import tpu as pltpu

RADIUS = 2
TOP_K = 4096
KS = 2 * RADIUS + 1
TEMP = 0.1
H = 512
W = 512
B = 4


def _mp5(x):
    h, w = x.shape
    minf_c = jnp.full((h, RADIUS), -jnp.inf, x.dtype)
    xp = jnp.concatenate([minf_c, x, minf_c], axis=1)
    m = xp[:, 0:w]
    for i in range(1, KS):
        m = jnp.maximum(m, xp[:, i:i + w])
    minf_r = jnp.full((RADIUS, w), -jnp.inf, x.dtype)
    vp = jnp.concatenate([minf_r, m, minf_r], axis=0)
    m = vp[0:h, :]
    for i in range(1, KS):
        m = jnp.maximum(m, vp[i:i + h, :])
    return m


def _nms_body(x_ref, nms_ref):
    x = x_ref[0, 0]
    max_mask = x == _mp5(x)
    zeros = jnp.zeros_like(x)
    for _ in range(2):
        supp_mask = _mp5(max_mask.astype(x.dtype)) > 0
        supp_scores = jnp.where(supp_mask, zeros, x)
        new_max = (supp_scores == _mp5(supp_scores)) & (~supp_mask)
        max_mask = max_mask | new_max
    nms = jnp.where(max_mask, x, zeros)
    ri = jax.lax.broadcasted_iota(jnp.int32, x.shape, 0)
    ci = jax.lax.broadcasted_iota(jnp.int32, x.shape, 1)
    inb = (ri >= RADIUS) & (ri < H - RADIUS) & (ci >= RADIUS) & (ci < W - RADIUS)
    nms_ref[0] = jnp.where(inb, nms, zeros)


@jax.jit
def _nms_pallas(scores_map):
    return pl.pallas_call(
        _nms_body,
        grid=(B,),
        in_specs=[pl.BlockSpec((1, 1, H, W), lambda b: (b, 0, 0, 0))],
        out_specs=pl.BlockSpec((1, H, W), lambda b: (b, 0, 0)),
        out_shape=jax.ShapeDtypeStruct((B, H, W), scores_map.dtype),
    )(scores_map)


def _hw_grid_host(r, dt):
    import numpy as np
    x = np.linspace(-r, r, 2 * r + 1)
    ii, jj = np.meshgrid(x, x, indexing='ij')
    return jnp.asarray(
        np.stack([jj.reshape(-1), ii.reshape(-1)], axis=1), dtype=dt)


def _grid_sample(img, xy):
    h, w = img.shape
    px = (xy[:, 0] + 1.0) * 0.5 * (w - 1)
    py = (xy[:, 1] + 1.0) * 0.5 * (h - 1)
    x0 = jnp.floor(px); y0 = jnp.floor(py)
    x1 = x0 + 1.0; y1 = y0 + 1.0
    wx1 = px - x0; wx0 = 1.0 - wx1
    wy1 = py - y0; wy0 = 1.0 - wy1
    def gat(xi, yi):
        valid = (xi >= 0) & (xi <= w - 1) & (yi >= 0) & (yi <= h - 1)
        xc = jnp.clip(xi, 0, w - 1).astype(jnp.int32)
        yc = jnp.clip(yi, 0, h - 1).astype(jnp.int32)
        return jnp.where(valid, img[yc, xc], 0.0)
    return (gat(x0, y0) * wx0 * wy0 + gat(x1, y0) * wx1 * wy0
            + gat(x0, y1) * wx0 * wy1 + gat(x1, y1) * wx1 * wy1)


@jax.jit
def kernel(scores_map):
    dt = scores_map.dtype
    nms = _nms_pallas(scores_map)
    flat = nms.reshape(B, -1)
    x0 = nms[:, :, 0::4]
    x1 = nms[:, :, 1::4]
    x2 = nms[:, :, 2::4]
    x3 = nms[:, :, 3::4]
    pcount = ((x0 > 0).astype(jnp.int32) + (x1 > 0).astype(jnp.int32)
              + (x2 > 0).astype(jnp.int32) + (x3 > 0).astype(jnp.int32))
    ok = jnp.all(pcount <= 1) & jnp.all(
        pcount.reshape(B, -1).sum(axis=1) >= TOP_K)
    v = jnp.maximum(jnp.maximum(x0, x1), jnp.maximum(x2, x3))
    o = jnp.where(x0 == v, 0,
                  jnp.where(x1 == v, 1, jnp.where(x2 == v, 2, 3)))
    ri = jax.lax.broadcasted_iota(jnp.int32, v.shape, 1)
    ci = jax.lax.broadcasted_iota(jnp.int32, v.shape, 2)
    cand_idx = (ri * W + 4 * ci + o).reshape(B, -1)
    cand_val = v.reshape(B, -1)

    def _fast(_):
        _, tki = jax.lax.top_k(cand_val, TOP_K)
        return jnp.take_along_axis(cand_idx, tki, axis=1)

    indices = jax.lax.cond(
        ok, _fast, lambda f: jax.lax.top_k(f, TOP_K)[1], flat)

    r = RADIUS
    padded = jnp.pad(scores_map[:, 0], ((0, 0), (r, r), (r, r)))
    import numpy as np
    oy, ox = np.meshgrid(np.arange(KS), np.arange(KS), indexing='ij')
    oy = jnp.asarray(oy.reshape(-1)); ox = jnp.asarray(ox.reshape(-1))
    hw_grid = _hw_grid_host(r, dt)
    wh = jnp.array([W - 1, H - 1], dtype=dt)

    def per_image(padded_b, img_b, idx):
        rows = idx // W
        cols = idx % W
        patch_scores = padded_b[rows[:, None] + oy[None, :],
                                cols[:, None] + ox[None, :]]
        kp_nms = jnp.stack([cols.astype(dt), rows.astype(dt)], axis=1)
        max_v = patch_scores.max(axis=1)[:, None]
        x_exp = jnp.exp((patch_scores - max_v) / TEMP)
        ssum = x_exp.sum(axis=1)[:, None]
        xy_residual = (x_exp @ hw_grid) / ssum
        dist2 = jnp.sum(((hw_grid[None, :, :] - xy_residual[:, None, :]) / r) ** 2,
                        axis=-1)
        disp = (x_exp * dist2).sum(axis=1) / ssum[:, 0]
        kp = (kp_nms + xy_residual) / wh * 2.0 - 1.0
        sc = _grid_sample(img_b, kp)
        return kp, disp, sc

    return jax.vmap(per_image)(padded, scores_map[:, 0], indices)


# speedup vs baseline: 2.2683x; 1.1032x over previous
import functools

import jax
import jax.numpy as jnp
from jax.experimental import pallas as pl
from jax.experimental.pallas import tpu as pltpu

RADIUS = 2
TOP_K = 4096
KS = 2 * RADIUS + 1
TEMP = 0.1
H = 512
W = 512
B = 4


def _mp5(x):
    h, w = x.shape
    minf_c = jnp.full((h, RADIUS), -jnp.inf, x.dtype)
    xp = jnp.concatenate([minf_c, x, minf_c], axis=1)
    m = xp[:, 0:w]
    for i in range(1, KS):
        m = jnp.maximum(m, xp[:, i:i + w])
    minf_r = jnp.full((RADIUS, w), -jnp.inf, x.dtype)
    vp = jnp.concatenate([minf_r, m, minf_r], axis=0)
    m = vp[0:h, :]
    for i in range(1, KS):
        m = jnp.maximum(m, vp[i:i + h, :])
    return m


def _nms_body(x_ref, nms_ref):
    x = x_ref[0, 0]
    max_mask = x == _mp5(x)
    zeros = jnp.zeros_like(x)
    for _ in range(2):
        supp_mask = _mp5(max_mask.astype(x.dtype)) > 0
        supp_scores = jnp.where(supp_mask, zeros, x)
        new_max = (supp_scores == _mp5(supp_scores)) & (~supp_mask)
        max_mask = max_mask | new_max
    nms = jnp.where(max_mask, x, zeros)
    ri = jax.lax.broadcasted_iota(jnp.int32, x.shape, 0)
    ci = jax.lax.broadcasted_iota(jnp.int32, x.shape, 1)
    inb = (ri >= RADIUS) & (ri < H - RADIUS) & (ci >= RADIUS) & (ci < W - RADIUS)
    nms_ref[0] = jnp.where(inb, nms, zeros)


@jax.jit
def _nms_pallas(scores_map):
    return pl.pallas_call(
        _nms_body,
        grid=(B,),
        in_specs=[pl.BlockSpec((1, 1, H, W), lambda b: (b, 0, 0, 0))],
        out_specs=pl.BlockSpec((1, H, W), lambda b: (b, 0, 0)),
        out_shape=jax.ShapeDtypeStruct((B, H, W), scores_map.dtype),
    )(scores_map)


def _hw_grid_host(r, dt):
    import numpy as np
    x = np.linspace(-r, r, 2 * r + 1)
    ii, jj = np.meshgrid(x, x, indexing='ij')
    return jnp.asarray(
        np.stack([jj.reshape(-1), ii.reshape(-1)], axis=1), dtype=dt)


def _grid_sample(img, xy):
    h, w = img.shape
    px = (xy[:, 0] + 1.0) * 0.5 * (w - 1)
    py = (xy[:, 1] + 1.0) * 0.5 * (h - 1)
    x0 = jnp.floor(px); y0 = jnp.floor(py)
    x1 = x0 + 1.0; y1 = y0 + 1.0
    wx1 = px - x0; wx0 = 1.0 - wx1
    wy1 = py - y0; wy0 = 1.0 - wy1
    def gat(xi, yi):
        valid = (xi >= 0) & (xi <= w - 1) & (yi >= 0) & (yi <= h - 1)
        xc = jnp.clip(xi, 0, w - 1).astype(jnp.int32)
        yc = jnp.clip(yi, 0, h - 1).astype(jnp.int32)
        return jnp.where(valid, img[yc, xc], 0.0)
    return (gat(x0, y0) * wx0 * wy0 + gat(x1, y0) * wx1 * wy0
            + gat(x0, y1) * wx0 * wy1 + gat(x1, y1) * wx1 * wy1)


@jax.jit
def kernel(scores_map):
    dt = scores_map.dtype
    nms = _nms_pallas(scores_map)
    flat = nms.reshape(B, -1)
    _, indices = jax.lax.top_k(flat, TOP_K)

    r = RADIUS
    padded = jnp.pad(scores_map[:, 0], ((0, 0), (r, r), (r, r)))
    import numpy as np
    oy, ox = np.meshgrid(np.arange(KS), np.arange(KS), indexing='ij')
    oy = jnp.asarray(oy.reshape(-1)); ox = jnp.asarray(ox.reshape(-1))
    hw_grid = _hw_grid_host(r, dt)
    wh = jnp.array([W - 1, H - 1], dtype=dt)

    def per_image(padded_b, img_b, idx):
        rows = idx // W
        cols = idx % W
        patch_scores = padded_b[rows[:, None] + oy[None, :],
                                cols[:, None] + ox[None, :]]
        kp_nms = jnp.stack([cols.astype(dt), rows.astype(dt)], axis=1)
        max_v = patch_scores.max(axis=1)[:, None]
        x_exp = jnp.exp((patch_scores - max_v) / TEMP)
        ssum = x_exp.sum(axis=1)[:, None]
        xy_residual = (x_exp @ hw_grid) / ssum
        dist2 = jnp.sum(((hw_grid[None, :, :] - xy_residual[:, None, :]) / r) ** 2,
                        axis=-1)
        disp = (x_exp * dist2).sum(axis=1) / ssum[:, 0]
        kp = (kp_nms + xy_residual) / wh * 2.0 - 1.0
        sc = _grid_sample(img_b, kp)
        return kp, disp, sc

    return jax.vmap(per_image)(padded, scores_map[:, 0], indices)
